# baseline (speedup 1.0000x reference)
import jax
import jax.numpy as jnp
from jax.experimental import pallas as pl
from jax.experimental.pallas import tpu as pltpu

NUM_EXPERTS = 8
TOP_K = 2
CHUNK = 1024


def _moe_kernel(x_ref, gw_ref, gb_ref, ew_ref, eb_ref, out_ref,
                c_ref, xb_ref, wb_ref, acc_ref):
    e = pl.program_id(0)
    tb = pl.program_id(1)
    E = NUM_EXPERTS
    sl = pl.ds(tb * CHUNK, CHUNK)

    @pl.when(tb == 0)
    def _cast_w():
        wb_ref[...] = ew_ref[0].astype(jnp.bfloat16)

    @pl.when(e == 0)
    def _gating():
        xc = x_ref[...]
        logits = jax.lax.dot_general(
            xc, gw_ref[...], (((1,), (1,)), ((), ())),
            preferred_element_type=jnp.float32,
            precision=jax.lax.Precision.DEFAULT,
        ) + gb_ref[...]
        m = jnp.max(logits, axis=1, keepdims=True)
        ex = jnp.exp(logits - m)
        w = ex / jnp.sum(ex, axis=1, keepdims=True)
        col = jax.lax.broadcasted_iota(jnp.int32, w.shape, 1)
        rank = jnp.zeros(w.shape, jnp.int32)
        for ep in range(E):
            wp = w[:, ep:ep + 1]
            rank = rank + (wp > w).astype(jnp.int32)
            rank = rank + ((wp == w) & (ep < col)).astype(jnp.int32)
        c_ref[sl, :] = jnp.where(rank < TOP_K, w, 0.0)
        xb_ref[sl, :] = xc.astype(jnp.bfloat16)

    cc = c_ref[sl, :]
    ce = jnp.sum(
        jnp.where(
            jax.lax.broadcasted_iota(jnp.int32, cc.shape, 1) == e,
            cc, 0.0),
        axis=1, keepdims=True)
    y = jax.lax.dot_general(
        xb_ref[sl, :], wb_ref[...], (((1,), (1,)), ((), ())),
        preferred_element_type=jnp.float32,
        precision=jax.lax.Precision.DEFAULT,
    ) + eb_ref[0]
    contrib = ce * y

    @pl.when(e == 0)
    def _init():
        acc_ref[sl, :] = contrib

    @pl.when((e > 0) & (e < E - 1))
    def _acc():
        acc_ref[sl, :] += contrib

    @pl.when(e == E - 1)
    def _flush():
        out_ref[...] = acc_ref[sl, :] + contrib


def kernel(x, gate_W, gate_b, expert_W, expert_b):
    T, D = x.shape
    E = gate_W.shape[0]
    nb = T // CHUNK
    return pl.pallas_call(
        _moe_kernel,
        grid=(E, nb),
        in_specs=[
            pl.BlockSpec((CHUNK, D),
                         lambda e, tb: (jnp.where(e == 0, tb, 1), 0)),
            pl.BlockSpec((E, D), lambda e, tb: (0, 0)),
            pl.BlockSpec((1, E), lambda e, tb: (0, 0)),
            pl.BlockSpec((1, D, D), lambda e, tb: (e, 0, 0)),
            pl.BlockSpec((1, 1, D), lambda e, tb: (e, 0, 0)),
        ],
        out_specs=pl.BlockSpec(
            (CHUNK, D),
            lambda e, tb: (jnp.where(e == NUM_EXPERTS - 1, tb, 0), 0)),
        out_shape=jax.ShapeDtypeStruct((T, D), jnp.float32),
        scratch_shapes=[
            pltpu.VMEM((T, E), jnp.float32),
            pltpu.VMEM((T, D), jnp.bfloat16),
            pltpu.VMEM((D, D), jnp.bfloat16),
            pltpu.VMEM((T, D), jnp.float32),
        ],
        compiler_params=pltpu.CompilerParams(
            dimension_semantics=("arbitrary", "arbitrary"),
        ),
    )(x, gate_W, gate_b.reshape(1, E), expert_W, expert_b.reshape(E, 1, D))
